# speedup vs baseline: 2.4197x; 2.4197x over previous
import functools

import jax
import jax.numpy as jnp
from jax import lax
from jax.experimental import pallas as pl
from jax.experimental.pallas import tpu as pltpu
from jax.experimental.pallas import tpu_sc as plsc

_MAX_DEGREE = 512
_HIDDEN = 128
_N = 100000

_NC = 2
_NS = 16
_NW = _NC * _NS

_CHUNK = 448
_FULL = 7 * _CHUNK
_TAIL_BASE = 31 * _FULL + 6 * _CHUNK
_TAIL = _N - _TAIL_BASE


def _body(deg_hbm, table_hbm, out_hbm,
          idx_v, rows_v, idx_t, rows_t, gsem, wsem):
    c = lax.axis_index("c")
    s = lax.axis_index("s")
    wid = s * _NC + c
    base = wid * _FULL

    def do_chunk(off, idxb, rowsb, n):
        pltpu.sync_copy(deg_hbm.at[pl.ds(off, n)], idxb)
        for i in range(n // 16):
            sl = pl.ds(i * 16, 16)
            v = idxb[sl]
            idxb[sl] = jnp.minimum(jnp.maximum(v, 0), _MAX_DEGREE - 1)
        pltpu.async_copy(table_hbm.at[idxb], rowsb, gsem).wait()
        pltpu.async_copy(rowsb, out_hbm.at[pl.ds(off, n)], wsem).wait()

    for j in range(6):
        do_chunk(base + j * _CHUNK, idx_v, rows_v, _CHUNK)

    @pl.when(wid < _NW - 1)
    def _():
        do_chunk(base + 6 * _CHUNK, idx_v, rows_v, _CHUNK)

    @pl.when(wid == _NW - 1)
    def _():
        do_chunk(_TAIL_BASE, idx_t, rows_t, _TAIL)


@jax.jit
def _run(degrees, table):
    mesh = plsc.VectorSubcoreMesh(core_axis_name="c", subcore_axis_name="s")
    k = pl.kernel(
        _body,
        mesh=mesh,
        out_type=jax.ShapeDtypeStruct((_N, _HIDDEN), jnp.float32),
        scratch_types=[
            pltpu.VMEM((_CHUNK,), jnp.int32),
            pltpu.VMEM((_CHUNK, _HIDDEN), jnp.float32),
            pltpu.VMEM((_TAIL,), jnp.int32),
            pltpu.VMEM((_TAIL, _HIDDEN), jnp.float32),
            pltpu.SemaphoreType.DMA,
            pltpu.SemaphoreType.DMA,
        ],
    )
    return k(degrees, table)


def kernel(degrees, degree_embedding):
    return _run(degrees.astype(jnp.int32), degree_embedding)
